# baseline (speedup 1.0000x reference)
import jax
import jax.numpy as jnp
from jax import lax
from jax.experimental import pallas as pl
from jax.experimental.pallas import tpu as pltpu
from jax.experimental.pallas import tpu_sc as plsc

_NC, _NS = 2, 16
_NW = _NC * _NS
_CHUNK = 64
_DEPTH = 4


def kernel(inputs, w):
    batch = inputs.shape[0]
    n_dim = w.shape[1]
    n_chunks = batch // (_NW * _CHUNK)
    idx = inputs.astype(jnp.int32).reshape(batch // _CHUNK, _CHUNK)

    mesh = plsc.VectorSubcoreMesh(core_axis_name="c", subcore_axis_name="s")

    scratch = (
        [pltpu.VMEM((n_chunks, _CHUNK), jnp.int32)]
        + [pltpu.VMEM((_CHUNK, n_dim), jnp.float32) for _ in range(n_chunks)]
        + [pltpu.SemaphoreType.DMA for _ in range(2 * n_chunks)]
    )

    @pl.kernel(out_type=jax.ShapeDtypeStruct((batch, n_dim), w.dtype),
               mesh=mesh, scratch_types=scratch)
    def gather_kernel(w_hbm, i_hbm, o_hbm, idx_v, *bufs_and_sems):
        bufs = bufs_and_sems[:n_chunks]
        sems_g = bufs_and_sems[n_chunks:2 * n_chunks]
        sems_w = bufs_and_sems[2 * n_chunks:]

        wid = lax.axis_index("s") * _NC + lax.axis_index("c")
        row0 = wid * n_chunks
        base = row0 * _CHUNK

        pltpu.sync_copy(i_hbm.at[pl.ds(row0, n_chunks)], idx_v)

        gathers = [
            pltpu.async_copy(w_hbm.at[idx_v.at[c]], bufs[c], sems_g[c])
            for c in range(_DEPTH)
        ]
        writes = []
        for c in range(n_chunks):
            gathers[c].wait()
            if c + _DEPTH < n_chunks:
                gathers.append(
                    pltpu.async_copy(w_hbm.at[idx_v.at[c + _DEPTH]],
                                     bufs[c + _DEPTH], sems_g[c + _DEPTH]))
            writes.append(
                pltpu.async_copy(
                    bufs[c], o_hbm.at[pl.ds(base + c * _CHUNK, _CHUNK)],
                    sems_w[c]))
        for wr in writes:
            wr.wait()

    out = gather_kernel(w, idx)
    return out[:, :, None]


# speedup vs baseline: 1.0493x; 1.0493x over previous
import jax
import jax.numpy as jnp
from jax import lax
from jax.experimental import pallas as pl
from jax.experimental.pallas import tpu as pltpu
from jax.experimental.pallas import tpu_sc as plsc

_NC, _NS = 2, 16
_NW = _NC * _NS
_CHUNK = 128


def kernel(inputs, w):
    batch = inputs.shape[0]
    n_dim = w.shape[1]
    n_chunks = batch // (_NW * _CHUNK)
    idx = inputs.astype(jnp.int32).reshape(1, batch)

    mesh = plsc.VectorSubcoreMesh(core_axis_name="c", subcore_axis_name="s")

    scratch = (
        [pltpu.VMEM((n_chunks, _CHUNK), jnp.int32),
         pltpu.VMEM((n_chunks * _CHUNK, n_dim), jnp.float32)]
        + [pltpu.SemaphoreType.DMA for _ in range(2 * n_chunks + 1)]
    )

    @pl.kernel(out_type=jax.ShapeDtypeStruct((batch, n_dim), w.dtype),
               mesh=mesh, scratch_types=scratch)
    def gather_kernel(w_hbm, i_hbm, o_hbm, idx_v, buf, *sems):
        sems_g = sems[:n_chunks]
        sems_i = sems[n_chunks:2 * n_chunks]
        sem_w = sems[2 * n_chunks]

        wid = lax.axis_index("s") * _NC + lax.axis_index("c")
        base = wid * n_chunks * _CHUNK

        idx_loads = [
            pltpu.async_copy(
                i_hbm.at[pl.ds(0, 1), pl.ds(base + c * _CHUNK, _CHUNK)],
                idx_v.at[pl.ds(c, 1)], sems_i[c])
            for c in range(n_chunks)
        ]
        gathers = []
        for c in range(n_chunks):
            idx_loads[c].wait()
            gathers.append(
                pltpu.async_copy(w_hbm.at[idx_v.at[c]],
                                 buf.at[pl.ds(c * _CHUNK, _CHUNK)],
                                 sems_g[c]))
        for g in gathers:
            g.wait()
        pltpu.async_copy(buf, o_hbm.at[pl.ds(base, n_chunks * _CHUNK)],
                         sem_w).wait()

    out = gather_kernel(w, idx)
    return out[:, :, None]


# speedup vs baseline: 1.0518x; 1.0024x over previous
import jax
import jax.numpy as jnp
from jax import lax
from jax.experimental import pallas as pl
from jax.experimental.pallas import tpu as pltpu
from jax.experimental.pallas import tpu_sc as plsc

_NC, _NS = 2, 16
_NW = _NC * _NS
_CHUNK = 128


def kernel(inputs, w):
    batch = inputs.shape[0]
    n_dim = w.shape[1]
    n_chunks = batch // (_NW * _CHUNK)
    idx = inputs.astype(jnp.int32).reshape(batch // _CHUNK, _CHUNK)

    mesh = plsc.VectorSubcoreMesh(core_axis_name="c", subcore_axis_name="s")

    scratch = (
        [pltpu.VMEM((n_chunks, _CHUNK), jnp.int32),
         pltpu.VMEM((n_chunks * _CHUNK, n_dim), jnp.float32)]
        + [pltpu.SemaphoreType.DMA for _ in range(n_chunks + 1)]
    )

    @pl.kernel(out_type=jax.ShapeDtypeStruct((batch, n_dim), w.dtype),
               mesh=mesh, scratch_types=scratch)
    def gather_kernel(w_hbm, i_hbm, o_hbm, idx_v, buf, *sems):
        sems_g = sems[:n_chunks]
        sem_w = sems[n_chunks]

        wid = lax.axis_index("s") * _NC + lax.axis_index("c")
        row0 = wid * n_chunks
        base = row0 * _CHUNK

        pltpu.sync_copy(i_hbm.at[pl.ds(row0, n_chunks)], idx_v)

        gathers = [
            pltpu.async_copy(w_hbm.at[idx_v.at[c]],
                             buf.at[pl.ds(c * _CHUNK, _CHUNK)], sems_g[c])
            for c in range(n_chunks)
        ]
        for g in gathers:
            g.wait()
        pltpu.async_copy(buf, o_hbm.at[pl.ds(base, n_chunks * _CHUNK)],
                         sem_w).wait()

    out = gather_kernel(w, idx)
    return out[:, :, None]
